# Initial kernel scaffold; baseline (speedup 1.0000x reference)
import jax
import jax.numpy as jnp
from jax.experimental import pallas as pl


def kernel(inputs, weight):
    raise NotImplementedError("write your pallas kernel here")


# speedup vs baseline: 1.3440x; 1.3440x over previous
import functools

import jax
import jax.numpy as jnp
from jax import lax
from jax.experimental import pallas as pl
from jax.experimental.pallas import tpu as pltpu
from jax.experimental.pallas import tpu_sc as plsc

_EPS = 1e-5
_BOUNDARY = 1.0 - _EPS
_VOCAB = 1000000
_DIM = 32
_BATCH = 16384
_SAMPLES = 50

_NC, _NS = 2, 16
_NW = _NC * _NS
_R = _BATCH * _SAMPLES
_RPW = _R // _NW
_CH = 1024
_NCH = _RPW // _CH


def _sc_gather_body(table_hbm, idx_hbm, out_hbm, idx_v, rows_v, sem):
    wid = lax.axis_index("s") * _NC + lax.axis_index("c")
    base = wid * _RPW
    for c in range(_NCH):
        off = base + c * _CH
        pltpu.sync_copy(idx_hbm.at[pl.ds(off, _CH)], idx_v)
        pltpu.async_copy(table_hbm.at[idx_v], rows_v, sem).wait()
        pltpu.sync_copy(rows_v, out_hbm.at[pl.ds(off, _CH)])


@functools.cache
def _sc_gather():
    return pl.kernel(
        _sc_gather_body,
        out_type=jax.ShapeDtypeStruct((_R, _DIM), jnp.float32),
        mesh=plsc.VectorSubcoreMesh(
            core_axis_name="c", subcore_axis_name="s",
            num_cores=_NC, num_subcores=_NS,
        ),
        scratch_types=[
            pltpu.VMEM((_CH,), jnp.int32),
            pltpu.VMEM((_CH, _DIM), jnp.float32),
            pltpu.SemaphoreType.DMA,
        ],
        compiler_params=pltpu.CompilerParams(use_tc_tiling_on_sc=False),
    )

_BB = 128


def _dist_body(e_ref, out_ref):
    e = e_ref[...]
    n = jnp.sqrt(jnp.sum(e * e, axis=-1, keepdims=True))
    scale = jnp.where(n > 1.0, 1.0 / (n + 1e-7), 1.0)
    e = e * scale
    sq = jnp.clip(jnp.sum(e * e, axis=-1), 0.0, _BOUNDARY)
    u = e[:, :1, :]
    o = e[:, 1:, :]
    sqdist = jnp.sum(jnp.square(u - o), axis=-1)
    squ = sq[:, :1]
    sqv = sq[:, 1:]
    x = sqdist / ((1.0 - squ) * (1.0 - sqv)) * 2.0 + 1.0
    z = jnp.sqrt(jnp.maximum(x * x - 1.0, 1e-12))
    out_ref[...] = -jnp.log(x + z)


_dist = pl.pallas_call(
    _dist_body,
    grid=(_BATCH // _BB,),
    in_specs=[pl.BlockSpec((_BB, _SAMPLES, _DIM), lambda i: (i, 0, 0))],
    out_specs=pl.BlockSpec((_BB, _SAMPLES - 1), lambda i: (i, 0)),
    out_shape=jax.ShapeDtypeStruct((_BATCH, _SAMPLES - 1), jnp.float32),
)


def kernel(inputs, weight):
    idx = inputs.reshape(-1)
    rows = _sc_gather()(weight, idx)
    e = rows.reshape(_BATCH, _SAMPLES, _DIM)
    return _dist(e)
